# Initial kernel scaffold; baseline (speedup 1.0000x reference)
import jax
import jax.numpy as jnp
from jax.experimental import pallas as pl


def kernel(x, edge_index, W_in1, b_in1, W_out1, b_out1, W_in2, b_in2, W_out2, b_out2):
    raise NotImplementedError("write your pallas kernel here")


# speedup vs baseline: 3.4962x; 3.4962x over previous
import functools

import jax
import jax.numpy as jnp
from jax import lax
from jax.experimental import pallas as pl
from jax.experimental.pallas import tpu as pltpu
from jax.experimental.pallas import tpu_sc as plsc

N = 10000
E = 320000
D = 128
DH = D // 2
ALPHA = 0.5

NC = 2
NS = 16
CHUNK = 128
N_PAD = 10240
E_PAD = 327680
NCHUNKS = E_PAD // CHUNK
CPT = NCHUNKS // NS
RPT = N_PAD // NS
CW = 16


def _make_sc_agg(with_counts):
    out_type = [jax.ShapeDtypeStruct((2, NC, N_PAD, DH), jnp.float32)]
    scratch = [
        pltpu.VMEM((CPT, CHUNK), jnp.int32),
        pltpu.VMEM((CPT, CHUNK), jnp.int32),
        pltpu.VMEM((CHUNK, DH), jnp.float32),
        pltpu.VMEM((CHUNK, DH), jnp.float32),
        pltpu.VMEM((CHUNK, DH), jnp.float32),
        pltpu.VMEM_SHARED((N_PAD, DH), jnp.float32),
        pltpu.SemaphoreType.DMA,
        pltpu.SemaphoreType.DMA,
    ]
    if with_counts:
        out_type.append(jax.ShapeDtypeStruct((NC, N_PAD, CW), jnp.float32))
        scratch += [
            pltpu.VMEM((CHUNK, CW), jnp.float32),
            pltpu.VMEM((CHUNK, CW), jnp.float32),
            pltpu.VMEM_SHARED((N_PAD, CW), jnp.float32),
        ]

    def body(*refs):
        if with_counts:
            (x0_hbm, x1_hbm, e_hbm, agg_hbm, cnt_hbm,
             gidx_v, sidx_v, rows0, rows1, zbuf, acc_sh, sem0, sem1,
             ones_v, cbuf, cnt_sh) = refs
        else:
            (x0_hbm, x1_hbm, e_hbm, agg_hbm,
             gidx_v, sidx_v, rows0, rows1, zbuf, acc_sh, sem0, sem1) = refs
            ones_v = cbuf = cnt_sh = cnt_hbm = None
        rows = (rows0, rows1)
        sems = (sem0, sem1)

        ci = lax.axis_index("c")
        si = lax.axis_index("s")
        base = si * RPT

        def init_body(i, carry):
            for k in range(DH // 16):
                zbuf[i, pl.ds(16 * k, 16)] = jnp.zeros((16,), jnp.float32)
            if with_counts:
                ones_v[i, pl.ds(0, 16)] = jnp.ones((16,), jnp.float32)
                cbuf[i, pl.ds(0, 16)] = jnp.zeros((16,), jnp.float32)
            return carry
        lax.fori_loop(0, CHUNK, init_body, 0)

        pltpu.sync_copy(e_hbm.at[ci, pl.ds(si * CPT, CPT)], gidx_v)
        pltpu.sync_copy(e_hbm.at[1 - ci, pl.ds(si * CPT, CPT)], sidx_v)

        for f in range(2):
            xf_hbm = x0_hbm if f == 0 else x1_hbm
            counts = with_counts and f == 0

            for k in range(RPT // CHUNK):
                sl = pl.ds(base + k * CHUNK, CHUNK)
                pltpu.sync_copy(zbuf, acc_sh.at[sl])
                if counts:
                    pltpu.sync_copy(cbuf, cnt_sh.at[sl])
            plsc.subcore_barrier()

            for b in range(2):
                pltpu.async_copy(xf_hbm.at[gidx_v.at[b]], rows[b], sems[b])

            def chunk_body(jj, carry):
                for b in range(2):
                    jb = 2 * jj + b
                    pltpu.make_async_copy(xf_hbm.at[pl.ds(0, CHUNK)], rows[b],
                                          sems[b]).wait()
                    pltpu.sync_copy(rows[b], acc_sh.at[sidx_v.at[jb]],
                                    add=True)
                    if counts:
                        pltpu.sync_copy(ones_v, cnt_sh.at[sidx_v.at[jb]],
                                        add=True)

                    @pl.when(jb + 2 < CPT)
                    def _():
                        pltpu.async_copy(xf_hbm.at[gidx_v.at[jb + 2]],
                                         rows[b], sems[b])
                return carry
            lax.fori_loop(0, CPT // 2, chunk_body, 0)
            plsc.subcore_barrier()

            for k in range(RPT // CHUNK):
                sl = pl.ds(base + k * CHUNK, CHUNK)
                pltpu.sync_copy(acc_sh.at[sl], rows0)
                pltpu.sync_copy(rows0, agg_hbm.at[f, ci, sl])
                if counts:
                    pltpu.sync_copy(cnt_sh.at[sl], cbuf)
                    pltpu.sync_copy(cbuf, cnt_hbm.at[ci, sl])

    mesh = plsc.VectorSubcoreMesh(core_axis_name="c", subcore_axis_name="s")
    return pl.kernel(body, out_type=tuple(out_type), mesh=mesh,
                     scratch_types=scratch,
                     compiler_params=pltpu.CompilerParams(
                         use_tc_tiling_on_sc=False))


def _tc_layer_body(relu, split_out, agg_ref, cnt_ref, wi_ref, wo_ref, b_ref,
                   *o_refs):
    a_in = jnp.concatenate([agg_ref[0, 0], agg_ref[1, 0]], axis=1)
    a_out = jnp.concatenate([agg_ref[0, 1], agg_ref[1, 1]], axis=1)
    c_in = cnt_ref[0, :, 0:1]
    c_out = cnt_ref[1, :, 0:1]
    m_in = a_in * ((1.0 - ALPHA) / jnp.maximum(c_in, 1.0))
    m_out = a_out * (ALPHA / jnp.maximum(c_out, 1.0))
    r = (jnp.dot(m_in, wi_ref[...], preferred_element_type=jnp.float32)
         + jnp.dot(m_out, wo_ref[...], preferred_element_type=jnp.float32)
         + b_ref[...])
    if relu:
        r = jnp.maximum(r, 0.0)
    if split_out:
        o_refs[0][...] = r[:, :DH]
        o_refs[1][...] = r[:, DH:]
    else:
        o_refs[0][...] = r


def _tc_layer(agg, cnt, w_in, w_out, b_comb, relu, split_out, nrows, blk):
    grid = (nrows // blk,)
    if split_out:
        out_specs = [pl.BlockSpec((blk, DH), lambda i: (i, 0)),
                     pl.BlockSpec((blk, DH), lambda i: (i, 0))]
        out_shape = [jax.ShapeDtypeStruct((nrows, DH), jnp.float32),
                     jax.ShapeDtypeStruct((nrows, DH), jnp.float32)]
    else:
        out_specs = pl.BlockSpec((blk, D), lambda i: (i, 0))
        out_shape = jax.ShapeDtypeStruct((nrows, D), jnp.float32)
    return pl.pallas_call(
        functools.partial(_tc_layer_body, relu, split_out),
        grid=grid,
        in_specs=[
            pl.BlockSpec((2, NC, blk, DH), lambda i: (0, 0, i, 0)),
            pl.BlockSpec((NC, blk, CW), lambda i: (0, i, 0)),
            pl.BlockSpec((D, D), lambda i: (0, 0)),
            pl.BlockSpec((D, D), lambda i: (0, 0)),
            pl.BlockSpec((1, D), lambda i: (0, 0)),
        ],
        out_specs=out_specs,
        out_shape=out_shape,
    )(agg, cnt, w_in, w_out, b_comb)


def kernel(x, edge_index, W_in1, b_in1, W_out1, b_out1,
           W_in2, b_in2, W_out2, b_out2):
    zpad = jnp.zeros((N_PAD - N, DH), jnp.float32)
    x0 = jnp.concatenate([x[:, :DH], zpad], axis=0)
    x1 = jnp.concatenate([x[:, DH:], zpad], axis=0)
    pad_edges = jnp.full((2, E_PAD - E), N, jnp.int32)
    edges = jnp.concatenate([edge_index, pad_edges], axis=1)
    edges = edges.reshape(2, NCHUNKS, CHUNK)

    b1 = ((1.0 - ALPHA) * b_in1 + ALPHA * b_out1).reshape(1, D)
    b2 = ((1.0 - ALPHA) * b_in2 + ALPHA * b_out2).reshape(1, D)

    sc_agg_counts = _make_sc_agg(True)
    sc_agg = _make_sc_agg(False)

    agg1, cnt = sc_agg_counts(x0, x1, edges)
    h0, h1 = _tc_layer(agg1, cnt, W_in1, W_out1, b1, True, True, N_PAD, 1024)
    (agg2,) = sc_agg(h0, h1, edges)
    out = _tc_layer(agg2, cnt, W_in2, W_out2, b2, False, False, N, 1000)
    return out
